# baseline (speedup 1.0000x reference)
import functools

import jax
import jax.numpy as jnp
from jax import lax
from jax.experimental import pallas as pl
from jax.experimental.pallas import tpu as pltpu
from jax.experimental.pallas import tpu_sc as plsc

S = 4096
E = 64
D = 4096
CAP = 64

R1 = 512
NB = S // R1
NC = 2
NS = 16
NW = NC * NS
TPW = S // NW
GPW = TPW // 16


def _stage1_kernel(x_ref, w_ref, idx_ref, gmax_ref, locw_ref, bcnt_ref,
                   me_ref, me_acc):
    i = pl.program_id(0)

    @pl.when(i == 0)
    def _init():
        me_acc[...] = jnp.zeros_like(me_acc)

    x = x_ref[...]
    w = w_ref[...]
    logits = jax.lax.dot_general(
        w, x, (((1,), (1,)), ((), ())), preferred_element_type=jnp.float32)
    rm = jnp.max(logits, axis=0, keepdims=True)
    unn = jnp.exp(logits - rm)
    den = jnp.sum(unn, axis=0, keepdims=True)
    gates = unn / den

    gmax = jnp.max(gates, axis=0, keepdims=True)
    rows = jax.lax.broadcasted_iota(jnp.int32, (E, R1), 0)
    idx = jnp.min(jnp.where(gates == gmax, rows, E), axis=0, keepdims=True)

    maskf = (rows == idx).astype(jnp.float32)
    ri = jax.lax.broadcasted_iota(jnp.int32, (R1, R1), 0)
    ci = jax.lax.broadcasted_iota(jnp.int32, (R1, R1), 1)
    tri = (ri <= ci).astype(jnp.float32)
    csum = jax.lax.dot_general(
        maskf, tri, (((1,), (0,)), ((), ())), preferred_element_type=jnp.float32)
    locw = jnp.sum((csum - 1.0) * maskf, axis=0, keepdims=True)

    ones_row = jnp.ones((1, R1), jnp.float32)
    bcnt = jax.lax.dot_general(
        ones_row, maskf, (((1,), (1,)), ((), ())),
        preferred_element_type=jnp.float32)

    me_acc[...] = me_acc[...] + jnp.sum(gates, axis=1, keepdims=True)

    idx_ref[...] = idx
    gmax_ref[...] = gmax
    locw_ref[...] = locw.astype(jnp.int32)
    bcnt_ref[...] = bcnt.astype(jnp.int32)[None]

    @pl.when(i == pl.num_programs(0) - 1)
    def _fin():
        me_ref[...] = me_acc[...]


def _route_kernel(idx_hbm, locw_hbm, bcnt_hbm, loc_hbm, cnt_hbm,
                  idx_v, locw_v, loc_v, bcnt_v, tot_v):
    c = lax.axis_index("c")
    s_ = lax.axis_index("s")
    w = s_ * NC + c
    base = w * TPW
    blk = base // R1

    pltpu.sync_copy(idx_hbm.at[pl.ds(base, TPW)], idx_v)
    pltpu.sync_copy(locw_hbm.at[pl.ds(base, TPW)], locw_v)
    pltpu.sync_copy(bcnt_hbm, bcnt_v)

    prefs, tots = [], []
    for k in range(E // 16):
        pref = jnp.zeros((16,), jnp.int32)
        tot = jnp.zeros((16,), jnp.int32)
        for b in range(NB):
            row = bcnt_v[b, pl.ds(k * 16, 16)]
            gate = jnp.where(b < blk, 1, 0)
            pref = pref + row * gate
            tot = tot + row
        prefs.append(pref)
        tots.append(tot)

    for g in range(GPW):
        sl = pl.ds(g * 16, 16)
        v = idx_v[sl]
        p = jnp.zeros((16,), jnp.int32)
        for k in range(E // 16):
            ii = jnp.clip(v - 16 * k, 0, 15)
            gk = prefs[k].at[ii].get(mode="promise_in_bounds")
            m = (v >= 16 * k) & (v < 16 * k + 16)
            p = jnp.where(m, gk, p)
        loc_v[sl] = locw_v[sl] + p

    pltpu.sync_copy(loc_v, loc_hbm.at[pl.ds(base, TPW)])

    @pl.when(w == 0)
    def _tot():
        for k in range(E // 16):
            tot_v[pl.ds(k * 16, 16)] = tots[k]
        pltpu.sync_copy(tot_v, cnt_hbm)


def _stage2_kernel(idx_ref, gmax_ref, loc_ref, me_ref, cnt_ref,
                   combine_ref, dispatch_ref, laux_ref, *, r):
    i = pl.program_id(0)
    idx = idx_ref[...]
    gmax = gmax_ref[...]
    loc = loc_ref[...]

    keep = loc < CAP
    loc_kept = jnp.where(keep, loc, -1)
    idx_k = jnp.where(keep, idx, -1)
    g1 = jnp.where(keep, gmax, 0.0)

    e3 = jax.lax.broadcasted_iota(jnp.int32, (E, CAP, r), 0)
    c3 = jax.lax.broadcasted_iota(jnp.int32, (E, CAP, r), 1)
    m3 = (e3 == idx_k[:, None, :]) & (c3 == loc_kept[:, None, :])
    combine_ref[...] = jnp.where(m3, g1[:, None, :], 0.0)
    dispatch_ref[...] = m3.astype(jnp.int8)

    @pl.when(i == 0)
    def _fin():
        me = me_ref[...]
        cnt = cnt_ref[...].astype(jnp.float32)
        laux_ref[...] = (jnp.sum(me * cnt, axis=0, keepdims=True)
                         * (float(E) / (float(S) * float(S))))


def _route_call(idx_flat, locw_flat, bcnt):
    mesh = plsc.VectorSubcoreMesh(core_axis_name="c", subcore_axis_name="s")
    kfn = pl.kernel(
        _route_kernel,
        mesh=mesh,
        out_type=[
            jax.ShapeDtypeStruct((S,), jnp.int32),
            jax.ShapeDtypeStruct((E,), jnp.int32),
        ],
        scratch_types=[
            pltpu.VMEM((TPW,), jnp.int32),
            pltpu.VMEM((TPW,), jnp.int32),
            pltpu.VMEM((TPW,), jnp.int32),
            pltpu.VMEM((NB, E), jnp.int32),
            pltpu.VMEM((E,), jnp.int32),
        ],
    )
    return kfn(idx_flat, locw_flat, bcnt)


def kernel(input, W):
    out1 = pl.pallas_call(
        _stage1_kernel,
        grid=(NB,),
        in_specs=[
            pl.BlockSpec((R1, D), lambda i: (i, 0)),
            pl.BlockSpec((E, D), lambda i: (0, 0)),
        ],
        out_specs=[
            pl.BlockSpec((1, R1), lambda i: (0, i)),
            pl.BlockSpec((1, R1), lambda i: (0, i)),
            pl.BlockSpec((1, R1), lambda i: (0, i)),
            pl.BlockSpec((1, 1, E), lambda i: (i, 0, 0)),
            pl.BlockSpec((E, 1), lambda i: (0, 0)),
        ],
        out_shape=[
            jax.ShapeDtypeStruct((1, S), jnp.int32),
            jax.ShapeDtypeStruct((1, S), jnp.float32),
            jax.ShapeDtypeStruct((1, S), jnp.int32),
            jax.ShapeDtypeStruct((NB, 1, E), jnp.int32),
            jax.ShapeDtypeStruct((E, 1), jnp.float32),
        ],
        scratch_shapes=[pltpu.VMEM((E, 1), jnp.float32)],
    )(input, W)
    idx2, gmax2, locw2, bcnt, me = out1

    loc_flat, cnt = _route_call(idx2.reshape(S), locw2.reshape(S),
                                bcnt.reshape(NB, E))

    r2 = 512
    out2 = pl.pallas_call(
        functools.partial(_stage2_kernel, r=r2),
        grid=(S // r2,),
        in_specs=[
            pl.BlockSpec((1, r2), lambda i: (0, i)),
            pl.BlockSpec((1, r2), lambda i: (0, i)),
            pl.BlockSpec((1, r2), lambda i: (0, i)),
            pl.BlockSpec((E, 1), lambda i: (0, 0)),
            pl.BlockSpec((E, 1), lambda i: (0, 0)),
        ],
        out_specs=[
            pl.BlockSpec((E, CAP, r2), lambda i: (0, 0, i)),
            pl.BlockSpec((E, CAP, r2), lambda i: (0, 0, i)),
            pl.BlockSpec((1, 1), lambda i: (0, 0)),
        ],
        out_shape=[
            jax.ShapeDtypeStruct((E, CAP, S), jnp.float32),
            jax.ShapeDtypeStruct((E, CAP, S), jnp.int8),
            jax.ShapeDtypeStruct((1, 1), jnp.float32),
        ],
    )(idx2, gmax2, loc_flat.reshape(1, S), me, cnt.reshape(E, 1))
    combine_t, dispatch_t, laux = out2

    combine = jnp.transpose(combine_t, (2, 0, 1))
    dispatch = jnp.transpose(dispatch_t != 0, (2, 0, 1))
    return (laux[0, 0], combine, dispatch, idx2.reshape(S), loc_flat,
            gmax2.reshape(S))


# speedup vs baseline: 1.3256x; 1.3256x over previous
import functools

import jax
import jax.numpy as jnp
from jax.experimental import pallas as pl
from jax.experimental.pallas import tpu as pltpu

S = 4096
E = 64
D = 4096
CAP = 64


def _gate_kernel(x_ref, w_ref, combine_ref, dispatch_ref, idx_ref, loc_ref,
                 gate_ref, laux_ref, counts_ref, me_ref, *, r, nsteps):
    i = pl.program_id(0)

    @pl.when(i == 0)
    def _init():
        counts_ref[...] = jnp.zeros_like(counts_ref)
        me_ref[...] = jnp.zeros_like(me_ref)

    x = x_ref[...]
    w = w_ref[...]
    logits = jax.lax.dot_general(
        w, x, (((1,), (1,)), ((), ())), preferred_element_type=jnp.float32)
    rm = jnp.max(logits, axis=0, keepdims=True)
    unn = jnp.exp(logits - rm)
    den = jnp.sum(unn, axis=0, keepdims=True)
    gates = unn / den

    gmax = jnp.max(gates, axis=0, keepdims=True)
    rows = jax.lax.broadcasted_iota(jnp.int32, (E, r), 0)
    idx = jnp.min(jnp.where(gates == gmax, rows, E), axis=0, keepdims=True)

    maskf = (rows == idx).astype(jnp.float32)
    ri = jax.lax.broadcasted_iota(jnp.int32, (r, r), 0)
    ci = jax.lax.broadcasted_iota(jnp.int32, (r, r), 1)
    tri = (ri <= ci).astype(jnp.float32)
    csum = jax.lax.dot_general(
        maskf, tri, (((1,), (0,)), ((), ())), preferred_element_type=jnp.float32)

    counts = counts_ref[...]
    loc_all = csum - 1.0 + counts
    loc_tok = jnp.sum(loc_all * maskf, axis=0, keepdims=True)

    counts_ref[...] = counts + jnp.sum(maskf, axis=1, keepdims=True)
    me_ref[...] = me_ref[...] + jnp.sum(gates, axis=1, keepdims=True)

    keep = loc_tok < float(CAP)
    loc_i = loc_tok.astype(jnp.int32)
    loc_kept = jnp.where(keep, loc_i, -1)
    idx_k = jnp.where(keep, idx, -1)
    g1 = jnp.where(keep, gmax, 0.0)

    pos = idx_k * CAP + loc_kept
    e3 = jax.lax.broadcasted_iota(jnp.int32, (E, CAP, r), 0)
    c3 = jax.lax.broadcasted_iota(jnp.int32, (E, CAP, r), 1)
    m3 = (e3 * CAP + c3) == pos[:, None, :]
    combine_ref[...] = jnp.where(m3, g1[:, None, :], 0.0)
    dispatch_ref[...] = m3.astype(jnp.int8)

    idx_ref[...] = idx
    loc_ref[...] = loc_i
    gate_ref[...] = gmax

    @pl.when(i == nsteps - 1)
    def _fin():
        me = me_ref[...]
        cnt = counts_ref[...]
        laux_ref[...] = (jnp.sum(me * cnt, axis=0, keepdims=True)
                         * (float(E) / (float(S) * float(S))))


def kernel(input, W):
    r = 512
    nsteps = S // r
    grid = (nsteps,)
    out = pl.pallas_call(
        functools.partial(_gate_kernel, r=r, nsteps=nsteps),
        grid=grid,
        in_specs=[
            pl.BlockSpec((r, D), lambda i: (i, 0)),
            pl.BlockSpec((E, D), lambda i: (0, 0)),
        ],
        out_specs=[
            pl.BlockSpec((E, CAP, r), lambda i: (0, 0, i)),
            pl.BlockSpec((E, CAP, r), lambda i: (0, 0, i)),
            pl.BlockSpec((1, r), lambda i: (0, i)),
            pl.BlockSpec((1, r), lambda i: (0, i)),
            pl.BlockSpec((1, r), lambda i: (0, i)),
            pl.BlockSpec((1, 1), lambda i: (0, 0)),
        ],
        out_shape=[
            jax.ShapeDtypeStruct((E, CAP, S), jnp.float32),
            jax.ShapeDtypeStruct((E, CAP, S), jnp.int8),
            jax.ShapeDtypeStruct((1, S), jnp.int32),
            jax.ShapeDtypeStruct((1, S), jnp.int32),
            jax.ShapeDtypeStruct((1, S), jnp.float32),
            jax.ShapeDtypeStruct((1, 1), jnp.float32),
        ],
        scratch_shapes=[
            pltpu.VMEM((E, 1), jnp.float32),
            pltpu.VMEM((E, 1), jnp.float32),
        ],
    )(input, W)
    combine_t, dispatch_t, idx, loc, g1s, laux = out
    combine = jnp.transpose(combine_t, (2, 0, 1))
    dispatch = jnp.transpose(dispatch_t != 0, (2, 0, 1))
    return (laux[0, 0], combine, dispatch, idx.reshape(S), loc.reshape(S),
            g1s.reshape(S))
